# baseline (speedup 1.0000x reference)
import functools

import jax
import jax.numpy as jnp
from jax import lax
from jax.experimental import pallas as pl
from jax.experimental.pallas import tpu as pltpu
from jax.experimental.pallas import tpu_sc as plsc

D_PAD = 128
D_MODEL = 64
NUM_WORKERS = 32
CHUNK = 512


def _make_gather(n_idx: int):
    per_w = n_idx // NUM_WORKERS
    n_chunks = per_w // CHUNK
    n_pairs = n_chunks // 2
    mesh = plsc.VectorSubcoreMesh(core_axis_name="c", subcore_axis_name="s")

    @functools.partial(
        pl.kernel,
        out_type=jax.ShapeDtypeStruct((n_idx, 2, D_MODEL), jnp.float32),
        mesh=mesh,
        scratch_types=[
            pltpu.VMEM((per_w,), jnp.int32),
            pltpu.VMEM((CHUNK, D_MODEL), jnp.float32),
            pltpu.VMEM((CHUNK, D_MODEL), jnp.float32),
            pltpu.SemaphoreType.DMA,
            pltpu.SemaphoreType.DMA,
        ],
        compiler_params=pltpu.CompilerParams(use_tc_tiling_on_sc=False),
    )
    def gather_kernel(table_hbm, idx_hbm, out_hbm, idx_v, rows0, rows1,
                      gsem0, gsem1):
        wid = lax.axis_index("s") * 2 + lax.axis_index("c")
        base = wid * per_w
        pltpu.sync_copy(idx_hbm.at[pl.ds(base, per_w)], idx_v)

        def gather(i, buf, sem):
            return pltpu.async_copy(
                table_hbm.at[idx_v.at[pl.ds(i * CHUNK, CHUNK)]], buf, sem)

        def store(i, buf):
            pltpu.sync_copy(buf, out_hbm.at[pl.ds(base + i * CHUNK, CHUNK), 0])

        g0 = gather(0, rows0, gsem0)

        def body(p, carry):
            i0 = 2 * p
            gather(i0 + 1, rows1, gsem1)
            g0.wait()
            store(i0, rows0)

            @pl.when(p < n_pairs - 1)
            def _():
                gather(i0 + 2, rows0, gsem0)

            pltpu.make_async_copy(
                table_hbm.at[idx_v.at[pl.ds(0, CHUNK)]], rows1, gsem1).wait()
            store(i0 + 1, rows1)
            return carry

        lax.fori_loop(0, n_pairs, body, 0)

    return gather_kernel


def kernel(x, table):
    b, l = x.shape
    n_idx = b * l
    idx = x.reshape(-1).astype(jnp.int32) * 2
    tbl = jnp.pad(table, ((0, 0), (0, D_PAD - D_MODEL)))
    tbl2 = tbl.reshape(-1, D_MODEL)
    out = _make_gather(n_idx)(tbl2, idx)
    return out[:, 0].reshape(b, l, D_MODEL)


# speedup vs baseline: 3.5812x; 3.5812x over previous
import functools

import jax
import jax.numpy as jnp
from jax import lax
from jax.experimental import pallas as pl
from jax.experimental.pallas import tpu as pltpu
from jax.experimental.pallas import tpu_sc as plsc

D_PAD = 128
D_MODEL = 64
NUM_WORKERS = 32
CHUNK = 512


def _make_gather(n_idx: int):
    per_w = n_idx // NUM_WORKERS
    n_chunks = per_w // CHUNK
    n_pairs = n_chunks // 2
    mesh = plsc.VectorSubcoreMesh(core_axis_name="c", subcore_axis_name="s")

    @functools.partial(
        pl.kernel,
        out_type=jax.ShapeDtypeStruct((n_idx, D_PAD), jnp.float32),
        mesh=mesh,
        scratch_types=[
            pltpu.VMEM((per_w,), jnp.int32),
            pltpu.VMEM((CHUNK, D_MODEL), jnp.float32),
            pltpu.VMEM((CHUNK, D_MODEL), jnp.float32),
            pltpu.SemaphoreType.DMA,
            pltpu.SemaphoreType.DMA,
        ],
        compiler_params=pltpu.CompilerParams(use_tc_tiling_on_sc=False),
    )
    def gather_kernel(table_hbm, idx_hbm, out_hbm, idx_v, rows0, rows1,
                      gsem0, gsem1):
        wid = lax.axis_index("s") * 2 + lax.axis_index("c")
        base = wid * per_w
        pltpu.sync_copy(idx_hbm.at[pl.ds(base, per_w)], idx_v)

        def gather(i, buf, sem):
            return pltpu.async_copy(
                table_hbm.at[idx_v.at[pl.ds(i * CHUNK, CHUNK)]], buf, sem)

        def store(i, buf):
            pltpu.sync_copy(
                buf, out_hbm.at[pl.ds(base + i * CHUNK, CHUNK), pl.ds(0, D_MODEL)])

        g0 = gather(0, rows0, gsem0)

        def body(p, carry):
            i0 = 2 * p
            gather(i0 + 1, rows1, gsem1)
            g0.wait()
            store(i0, rows0)

            @pl.when(p < n_pairs - 1)
            def _():
                gather(i0 + 2, rows0, gsem0)

            pltpu.make_async_copy(
                table_hbm.at[idx_v.at[pl.ds(0, CHUNK)]], rows1, gsem1).wait()
            store(i0 + 1, rows1)
            return carry

        lax.fori_loop(0, n_pairs, body, 0)

    return gather_kernel


def kernel(x, table):
    b, l = x.shape
    n_idx = b * l
    idx = x.reshape(-1).astype(jnp.int32) * 2
    tbl = jnp.pad(table, ((0, 0), (0, D_PAD - D_MODEL)))
    tbl2 = tbl.reshape(-1, D_MODEL)
    out = _make_gather(n_idx)(tbl2, idx)
    return out.reshape(b, l, D_PAD)[:, :, :D_MODEL]
